# baseline (speedup 1.0000x reference)
import functools

import jax
import jax.numpy as jnp
from jax import lax
from jax.experimental import pallas as pl
from jax.experimental.pallas import tpu as pltpu
from jax.experimental.pallas import tpu_sc as plsc

_B = 16384
_D = 16
_NC = 2
_NS = 16
_NW = _NC * _NS
_BPW = _B // _NW
_CHUNK = 128
_NCHUNK = _BPW // _CHUNK


def _dot_kernel(table_hbm, cen_hbm, ctx_hbm, out_hbm,
                cen_idx, ctx_idx, cen_rows, ctx_rows, out_v, sem):
    wid = lax.axis_index("s") * _NC + lax.axis_index("c")
    base = wid * _BPW

    pltpu.sync_copy(cen_hbm.at[wid], cen_idx)
    pltpu.sync_copy(ctx_hbm.at[wid], ctx_idx)

    copies = []
    for j in range(_NCHUNK):
        dst = cen_rows.at[pl.ds(j * _CHUNK, _CHUNK)]
        copies.append(pltpu.async_copy(table_hbm.at[cen_idx.at[j]], dst, sem))
        dst = ctx_rows.at[pl.ds(j * _CHUNK, _CHUNK)]
        copies.append(pltpu.async_copy(table_hbm.at[ctx_idx.at[j]], dst, sem))
    for c in copies:
        c.wait()

    lane = lax.iota(jnp.int32, 16)

    def tile_body(t, _):
        acc = jnp.zeros((16,), jnp.float32)
        for r in range(16):
            i = t * 16 + r
            p = cen_rows[i] * ctx_rows[i]
            s = jnp.sum(p)
            acc = jnp.where(lane == r, s, acc)
        out_v[pl.ds(t * 16, 16)] = acc
        return ()

    lax.fori_loop(0, _BPW // 16, tile_body, ())

    pltpu.sync_copy(out_v, out_hbm.at[pl.ds(base, _BPW)])


@jax.jit
def kernel(inputs, table):
    cen = inputs[:, 0].reshape(_NW, _NCHUNK, _CHUNK)
    ctx = inputs[:, 1].reshape(_NW, _NCHUNK, _CHUNK)

    k = functools.partial(
        pl.kernel,
        mesh=plsc.VectorSubcoreMesh(core_axis_name="c", subcore_axis_name="s"),
        compiler_params=pltpu.CompilerParams(
            needs_layout_passes=False, use_tc_tiling_on_sc=False),
        out_type=jax.ShapeDtypeStruct((_B,), jnp.float32),
        scratch_types=[
            pltpu.VMEM((_NCHUNK, _CHUNK), jnp.int32),
            pltpu.VMEM((_NCHUNK, _CHUNK), jnp.int32),
            pltpu.VMEM((_BPW, _D), jnp.float32),
            pltpu.VMEM((_BPW, _D), jnp.float32),
            pltpu.VMEM((_BPW,), jnp.float32),
            pltpu.SemaphoreType.DMA,
        ],
    )(_dot_kernel)

    out = k(table, cen, ctx)
    return out.reshape(_B, 1)
